# baseline (speedup 1.0000x reference)
import functools

import jax
import jax.numpy as jnp
from jax import lax
from jax.experimental import pallas as pl
from jax.experimental.pallas import tpu as pltpu
from jax.experimental.pallas import tpu_sc as plsc

_NBUF = 2
_CHUNK = 128


def _prep_tc(pos_table, seg_table, pos2d, seg2d):
    num_seg, dim = seg_table.shape
    max_len = pos_table.shape[0]
    n_r, n_c = pos2d.shape

    def body(pos_ref, seg_ref, p2_ref, s2_ref, fus_ref, fidx_ref):
        fus_ref[...] = seg_ref[...][:, None, :] + pos_ref[...][None, :, :]
        fidx_ref[...] = s2_ref[...] * max_len + p2_ref[...]

    fused, fidx = pl.pallas_call(
        body,
        out_shape=(
            jax.ShapeDtypeStruct((num_seg, max_len, dim), jnp.float32),
            jax.ShapeDtypeStruct((n_r, n_c), jnp.int32),
        ),
    )(pos_table, seg_table, pos2d, seg2d)
    return fused.reshape(num_seg * max_len, dim), fidx


def _sc_lookup(n_rows, dim, n_fused):
    info = plsc.get_sparse_core_info()
    nc, ns, lanes = info.num_cores, info.num_subcores, info.num_lanes
    nw = nc * ns
    CHUNK = _CHUNK
    NBUF = _NBUF
    rows_per_w = n_rows // (nw * CHUNK)
    HALF = rows_per_w // 2
    mesh = plsc.VectorSubcoreMesh(core_axis_name="c", subcore_axis_name="s")

    @functools.partial(
        pl.kernel,
        mesh=mesh,
        out_type=jax.ShapeDtypeStruct((n_rows, dim), jnp.float32),
        scratch_types=(
            [pltpu.VMEM((HALF, CHUNK), jnp.int32)] * 2
            + [pltpu.VMEM((CHUNK, dim), jnp.float32)] * (3 * NBUF)
            + [pltpu.SemaphoreType.DMA] * (2 * NBUF)
        ),
    )
    def k(tok_hbm, fidx_hbm, toktab_hbm, fustab_hbm, out_hbm, *refs):
        tokidx, fidxv = refs[0], refs[1]
        bufs = refs[2:2 + 3 * NBUF]
        tokbuf = bufs[0:NBUF]
        fusbuf = bufs[NBUF:2 * NBUF]
        outbuf = bufs[2 * NBUF:3 * NBUF]
        sems = refs[2 + 3 * NBUF:]
        gsem = sems[0:NBUF]
        wsem = sems[NBUF:2 * NBUF]

        cid = lax.axis_index("c")
        sid = lax.axis_index("s")
        wid = sid * nc + cid
        rowbase = wid * rows_per_w

        HC = CHUNK // 2

        def fire_gathers(b, cg):
            for p in range(2):
                sl = pl.ds(p * HC, HC)
                pltpu.async_copy(toktab_hbm.at[tokidx.at[cg, sl]],
                                 tokbuf[b].at[pl.ds(p * HC, HC)], gsem[b])
                pltpu.async_copy(fustab_hbm.at[fidxv.at[cg, sl]],
                                 fusbuf[b].at[pl.ds(p * HC, HC)], gsem[b])

        def wait_gathers(b):
            pltpu.make_async_copy(toktab_hbm.at[pl.ds(0, CHUNK)], tokbuf[b],
                                  gsem[b]).wait()
            pltpu.make_async_copy(toktab_hbm.at[pl.ds(0, CHUNK)], fusbuf[b],
                                  gsem[b]).wait()

        def wait_write(b):
            pltpu.make_async_copy(outbuf[b], out_hbm.at[pl.ds(0, CHUNK)],
                                  wsem[b]).wait()

        for h in range(2):
            hb = rowbase + h * HALF
            pltpu.sync_copy(tok_hbm.at[pl.ds(hb, HALF)], tokidx)
            pltpu.sync_copy(fidx_hbm.at[pl.ds(hb, HALF)], fidxv)
            for b in range(NBUF):
                fire_gathers(b, b)

            def body(kk, carry):
                for b in range(NBUF):
                    cg = NBUF * kk + b
                    gidx = h * HALF + cg
                    wait_gathers(b)

                    @pl.when(gidx >= NBUF)
                    def _():
                        wait_write(b)

                    def addrow(r, acc):
                        for j in range(dim // lanes):
                            sl = pl.ds(j * lanes, lanes)
                            outbuf[b][r, sl] = (tokbuf[b][r, sl]
                                                + fusbuf[b][r, sl])
                        return acc

                    lax.fori_loop(0, CHUNK, addrow, 0, unroll=False)

                    @pl.when(cg + NBUF < HALF)
                    def _():
                        fire_gathers(b, cg + NBUF)

                    pltpu.async_copy(
                        outbuf[b],
                        out_hbm.at[pl.ds((rowbase + gidx) * CHUNK, CHUNK)],
                        wsem[b])
                return carry

            lax.fori_loop(0, HALF // NBUF, body, 0, unroll=False)

        for b in range(NBUF):
            wait_write(b)

    return k


def kernel(tokens, segment_ids, pos_ids, token_table, pos_table, seg_table):
    b, l = tokens.shape
    vocab, dim = token_table.shape
    max_len = pos_table.shape[0]
    num_seg = seg_table.shape[0]
    n_rows = b * l
    n_c = _CHUNK
    n_r = n_rows // n_c

    tok2d = tokens.reshape(n_r, n_c).astype(jnp.int32)
    pos2d = pos_ids.reshape(n_r, n_c).astype(jnp.int32)
    seg2d = segment_ids.reshape(n_r, n_c).astype(jnp.int32)

    fused, fidx2d = _prep_tc(pos_table, seg_table, pos2d, seg2d)

    out = _sc_lookup(n_rows, dim, num_seg * max_len)(
        tok2d, fidx2d, token_table, fused)
    return out.reshape(b, l, dim)
